# baseline (speedup 1.0000x reference)
import functools

import jax
import jax.numpy as jnp
from jax import lax
from jax.experimental import pallas as pl
from jax.experimental.pallas import tpu as pltpu
from jax.experimental.pallas import tpu_sc as plsc

_DP = 32
_GB = 4

_INFO = plsc.get_sparse_core_info()
_NC = _INFO.num_cores
_NS = _INFO.num_subcores
_NW = _NC * _NS


@functools.cache
def _build(B, S):
  rows_per_w = B // _NW
  ng = rows_per_w // _GB
  chunks = []
  s0 = 0
  while s0 < S:
    c = min(128, S - s0)
    chunks.append((s0, c))
    s0 += c
  mesh = plsc.VectorSubcoreMesh(core_axis_name="c", subcore_axis_name="s")

  @functools.partial(
      pl.kernel, mesh=mesh,
      out_type=jax.ShapeDtypeStruct((B, S, _DP), jnp.float32),
      compiler_params=pltpu.CompilerParams(use_tc_tiling_on_sc=False),
      scratch_types=[
          pltpu.VMEM((_GB, S), jnp.int32),
          pltpu.VMEM((_GB, S, _DP), jnp.float32),
          pltpu.SemaphoreType.DMA,
      ],
  )
  def gather_kernel(idx_hbm, table_hbm, out_hbm, idx_v, rows_v, sem):
    wid = lax.axis_index("s") * _NC + lax.axis_index("c")
    base = wid * rows_per_w

    def step(g, carry):
      b0 = base + g * _GB
      pltpu.sync_copy(idx_hbm.at[pl.ds(b0, _GB)], idx_v)
      copies = []
      for bi in range(_GB):
        for (s0, c) in chunks:
          copies.append(pltpu.async_copy(
              table_hbm.at[idx_v.at[bi, pl.ds(s0, c)]],
              rows_v.at[bi, pl.ds(s0, c)], sem))
      for cp in copies:
        cp.wait()
      pltpu.sync_copy(rows_v, out_hbm.at[pl.ds(b0, _GB)])
      return carry

    lax.fori_loop(0, ng, step, 0)

  return gather_kernel


def kernel(indices, table):
  B, S = indices.shape
  V, D = table.shape
  idx = indices.astype(jnp.int32)
  tab = jnp.pad(table.T, ((0, _DP - D), (0, 0))).T
  out = _build(B, S)(idx, tab)
  return out[:, :, :D]
